# baseline (speedup 1.0000x reference)
import functools

import jax
import jax.numpy as jnp
from jax import lax
from jax.experimental import pallas as pl
from jax.experimental.pallas import tpu as pltpu
from jax.experimental.pallas import tpu_sc as plsc

N_PAIRS = 320000
N_ATOMS = 10000
N_MOL = 512
D = 128
NC = 2
NS = 16
NW = NC * NS
PT = N_PAIRS // NW
L = 16
C = 80
NCHUNK = PT // C


def _sc_segment_sum(pairfeatures, mol_index, pair_first, zeros):
    mesh = plsc.VectorSubcoreMesh(core_axis_name="c", subcore_axis_name="s")

    @functools.partial(
        pl.kernel,
        mesh=mesh,
        out_type=jax.ShapeDtypeStruct((NC, N_MOL, D), jnp.float32),
        compiler_params=pltpu.CompilerParams(needs_layout_passes=False),
        scratch_types=[
            pltpu.VMEM((PT,), jnp.int32),
            pltpu.VMEM((N_ATOMS,), jnp.int32),
            pltpu.VMEM((NCHUNK, C), jnp.int32),
            *[pltpu.VMEM((C, D), jnp.float32) for _ in range(6)],
            pltpu.VMEM_SHARED((N_MOL, D), jnp.float32),
            *[pltpu.SemaphoreType.DMA for _ in range(12)],
        ],
    )
    def seg_sum(feat_hbm, mi_hbm, pf_hbm, z_hbm, out_hbm,
                pf_v, mi_v, pm_v, fv0, fv1, fv2, fv3, fv4, fv5, acc_sh,
                li0, li1, li2, li3, li4, li5, ai0, ai1, ai2, ai3, ai4, ai5):
        core = lax.axis_index("c")
        sub = lax.axis_index("s")
        wid = core * NS + sub
        base = wid * PT
        bufs = (fv0, fv1, fv2, fv3, fv4, fv5)
        lsems = (li0, li1, li2, li3, li4, li5)
        asems = (ai0, ai1, ai2, ai3, ai4, ai5)
        NB = 6

        def start_load(j, b):
            pltpu.async_copy(feat_hbm.at[pl.ds(base + j * C, C)], bufs[b], lsems[b])

        def wait_load(b):
            pltpu.make_async_copy(feat_hbm.at[pl.ds(0, C)], bufs[b], lsems[b]).wait()

        def start_add(j, b):
            pltpu.async_copy(bufs[b], acc_sh.at[pm_v.at[j]], asems[b], add=True)

        def wait_add(j, b):
            pltpu.make_async_copy(bufs[b], acc_sh.at[pm_v.at[j]], asems[b]).wait()

        for _j in range(5):
            start_load(_j, _j)

        pltpu.sync_copy(pf_hbm.at[pl.ds(base, PT)], pf_v)
        pltpu.sync_copy(mi_hbm, mi_v)

        @pl.when(sub == 0)
        def _():
            pltpu.sync_copy(z_hbm, acc_sh)

        def gather_body(j, carry):
            r0 = j * C
            for k in range(C // L):
                idx = pf_v[pl.ds(r0 + k * L, L)]
                pm_v[j, pl.ds(k * L, L)] = plsc.load_gather(mi_v, [idx])
            return carry

        lax.fori_loop(0, NCHUNK, gather_body, 0)

        plsc.subcore_barrier()

        def add_body(jj, carry):
            for b in range(NB):
                j = NB * jj + b
                wait_load(b)
                start_add(j, b)

                @pl.when(j >= 1)
                def _():
                    wait_add(j - 1, (b - 1) % NB)

                @pl.when(j + (NB - 1) < NCHUNK)
                def _():
                    start_load(j + (NB - 1), (b + NB - 1) % NB)

            return carry

        NFULL = NCHUNK // NB
        lax.fori_loop(0, NFULL, add_body, 0)

        for j in range(NFULL * NB, NCHUNK):
            b = j % NB
            wait_load(b)
            start_add(j, b)
            wait_add(j - 1, (b - 1) % NB)
        wait_add(NCHUNK - 1, (NCHUNK - 1) % NB)

        plsc.subcore_barrier()

        rows = N_MOL // NS
        pltpu.sync_copy(acc_sh.at[pl.ds(sub * rows, rows)],
                        out_hbm.at[core, pl.ds(sub * rows, rows)])

    return seg_sum(pairfeatures, mol_index, pair_first, zeros)


def _combine(partials):
    def body(p_ref, o_ref):
        o_ref[...] = p_ref[0] + p_ref[1]

    return pl.pallas_call(
        body,
        out_shape=jax.ShapeDtypeStruct((N_MOL, D), jnp.float32),
    )(partials)


def kernel(pairfeatures, mol_index, n_molecules, pair_first):
    zeros = jnp.zeros((N_MOL, D), dtype=jnp.float32)
    partials = _sc_segment_sum(pairfeatures,
                               mol_index.astype(jnp.int32),
                               pair_first.astype(jnp.int32),
                               zeros)
    return _combine(partials)
